# baseline (speedup 1.0000x reference)
import functools

import jax
import jax.numpy as jnp
from jax import lax
from jax.experimental import pallas as pl
from jax.experimental.pallas import tpu as pltpu
from jax.experimental.pallas import tpu_sc as plsc

D_OUT = 128
HALF = 64
GRID = 256
NC, NS = 2, 16
NW = NC * NS
SPOTS = 16 * 4096
PER_W = SPOTS // NW
CHUNK = 64
NCHUNK = PER_W // CHUNK
IDX_W = 2 * CHUNK


def _tables_body(row_ref, col_ref, w_ref, b_ref, t_ref):
    w = w_ref[...]
    tr = lax.dot_general(row_ref[...], w[:, :HALF],
                         (((1,), (1,)), ((), ())),
                         preferred_element_type=jnp.float32)
    t_ref[:GRID] = tr + b_ref[...]
    t_ref[GRID:] = lax.dot_general(col_ref[...], w[:, HALF:],
                                   (((1,), (1,)), ((), ())),
                                   preferred_element_type=jnp.float32)


def _make_table(row_embed, col_embed, w, b):
    return pl.pallas_call(
        _tables_body,
        out_shape=jax.ShapeDtypeStruct((2 * GRID, D_OUT), jnp.float32),
    )(row_embed, col_embed, w, b.reshape(1, D_OUT))


_sc_mesh = plsc.VectorSubcoreMesh(core_axis_name="c", subcore_axis_name="s")


@functools.partial(
    pl.kernel,
    out_type=jax.ShapeDtypeStruct((SPOTS, D_OUT), jnp.float32),
    mesh=_sc_mesh,
    scratch_types=[
        pltpu.VMEM((NCHUNK, IDX_W), jnp.int32),
        pltpu.VMEM((IDX_W, D_OUT), jnp.float32),
        pltpu.VMEM((IDX_W, D_OUT), jnp.float32),
        pltpu.VMEM((CHUNK, D_OUT), jnp.float32),
        pltpu.VMEM((CHUNK, D_OUT), jnp.float32),
        pltpu.VMEM_SHARED((2 * GRID, D_OUT), jnp.float32),
        pltpu.SemaphoreType.DMA,
        pltpu.SemaphoreType.DMA,
        pltpu.SemaphoreType.DMA,
        pltpu.SemaphoreType.DMA,
    ],
)
def _sc_lookup(t_hbm, coords_hbm, out_hbm,
               idx, buf0, buf1, bufo0, bufo1, t_sp,
               semg0, semg1, semo0, semo1):
    wid = lax.axis_index("s") * NC + lax.axis_index("c")
    base = wid * PER_W
    buf = (buf0, buf1)
    bufo = (bufo0, bufo1)
    semg = (semg0, semg1)
    semo = (semo0, semo1)
    @pl.when(lax.axis_index("s") == 0)
    def _stage():
        pltpu.sync_copy(t_hbm, t_sp)

    pltpu.sync_copy(coords_hbm.at[wid], idx)
    parity = (lax.iota(jnp.int32, 16) & 1) * GRID

    def _fix(j, _):
        for cpos in range(IDX_W // 16):
            sl = pl.ds(cpos * 16, 16)
            v = idx[j, sl]
            v = jnp.minimum(jnp.maximum(v, 0), GRID - 1)
            idx[j, sl] = v + parity
        return 0

    lax.fori_loop(0, NCHUNK, _fix, 0)
    plsc.subcore_barrier()
    g = [pltpu.async_copy(t_sp.at[idx.at[p]], buf[p], semg[p])
         for p in range(2)]
    oc = [None, None]
    for j in range(NCHUNK):
        p = j & 1
        g[p].wait()
        if oc[p] is not None:
            oc[p].wait()

        def _add_row(r, _, p=p):
            for c in range(D_OUT // 16):
                sl = pl.ds(c * 16, 16)
                bufo[p][r, sl] = buf[p][2 * r, sl] + buf[p][2 * r + 1, sl]
            return 0

        lax.fori_loop(0, CHUNK, _add_row, 0)
        oc[p] = pltpu.async_copy(
            bufo[p], out_hbm.at[pl.ds(base + j * CHUNK, CHUNK)], semo[p])
        if j + 2 < NCHUNK:
            g[p] = pltpu.async_copy(t_sp.at[idx.at[j + 2]], buf[p], semg[p])
    oc[0].wait()
    oc[1].wait()


def kernel(coords, row_embed, col_embed, W, b):
    batch, n_spots, _ = coords.shape
    table = _make_table(row_embed, col_embed, W, b)
    cflat = coords.astype(jnp.int32).reshape(NW, NCHUNK, IDX_W)
    out = _sc_lookup(table, cflat)
    return out.reshape(batch, n_spots, D_OUT)


# speedup vs baseline: 2.6844x; 2.6844x over previous
import functools

import jax
import jax.numpy as jnp
from jax import lax
from jax.experimental import pallas as pl
from jax.experimental.pallas import tpu as pltpu
from jax.experimental.pallas import tpu_sc as plsc

D_OUT = 128
HALF = 64
GRID = 256
NC, NS = 2, 16
NW = NC * NS
SPOTS = 16 * 4096
PER_W = SPOTS // NW
CHUNK = 128
NCHUNK = PER_W // CHUNK


def _tables_body(row_ref, col_ref, w_ref, b_ref, tr_ref, tc_ref):
    w = w_ref[...]
    tr = lax.dot_general(row_ref[...], w[:, :HALF],
                         (((1,), (1,)), ((), ())),
                         preferred_element_type=jnp.float32)
    tr_ref[...] = tr + b_ref[...]
    tc_ref[...] = lax.dot_general(col_ref[...], w[:, HALF:],
                                  (((1,), (1,)), ((), ())),
                                  preferred_element_type=jnp.float32)


def _make_tables(row_embed, col_embed, w, b):
    return pl.pallas_call(
        _tables_body,
        out_shape=(
            jax.ShapeDtypeStruct((GRID, D_OUT), jnp.float32),
            jax.ShapeDtypeStruct((GRID, D_OUT), jnp.float32),
        ),
    )(row_embed, col_embed, w, b.reshape(1, D_OUT))


_sc_mesh = plsc.VectorSubcoreMesh(core_axis_name="c", subcore_axis_name="s")


@functools.partial(
    pl.kernel,
    out_type=jax.ShapeDtypeStruct((SPOTS, D_OUT), jnp.float32),
    mesh=_sc_mesh,
    scratch_types=[
        pltpu.VMEM((NCHUNK, CHUNK), jnp.int32),
        pltpu.VMEM((NCHUNK, CHUNK), jnp.int32),
        pltpu.VMEM((CHUNK, D_OUT), jnp.float32),
        pltpu.VMEM((CHUNK, D_OUT), jnp.float32),
        pltpu.VMEM((CHUNK, D_OUT), jnp.float32),
        pltpu.VMEM((CHUNK, D_OUT), jnp.float32),
        pltpu.VMEM((CHUNK, D_OUT), jnp.float32),
        pltpu.VMEM((CHUNK, D_OUT), jnp.float32),
        pltpu.VMEM_SHARED((GRID, D_OUT), jnp.float32),
        pltpu.VMEM_SHARED((GRID, D_OUT), jnp.float32),
        pltpu.SemaphoreType.DMA,
        pltpu.SemaphoreType.DMA,
        pltpu.SemaphoreType.DMA,
        pltpu.SemaphoreType.DMA,
        pltpu.SemaphoreType.DMA,
        pltpu.SemaphoreType.DMA,
        pltpu.SemaphoreType.DMA,
        pltpu.SemaphoreType.DMA,
        pltpu.SemaphoreType.DMA,
    ],
)
def _sc_lookup(tr_hbm, tc_hbm, rows_hbm, cols_hbm, out_hbm,
               idxr, idxc, bufr0, bufr1, bufr2, bufc0, bufc1, bufc2,
               tr_sp, tc_sp, semr0, semr1, semr2, semc0, semc1, semc2,
               semo0, semo1, semo2):
    wid = lax.axis_index("s") * NC + lax.axis_index("c")
    base = wid * PER_W
    bufr = (bufr0, bufr1, bufr2)
    bufc = (bufc0, bufc1, bufc2)
    semr = (semr0, semr1, semr2)
    semc = (semc0, semc1, semc2)
    semo = (semo0, semo1, semo2)
    @pl.when(lax.axis_index("s") == 0)
    def _stage():
        pltpu.sync_copy(tr_hbm, tr_sp)
        pltpu.sync_copy(tc_hbm, tc_sp)

    pltpu.sync_copy(rows_hbm.at[wid], idxr)
    pltpu.sync_copy(cols_hbm.at[wid], idxc)
    plsc.subcore_barrier()
    gr = [pltpu.async_copy(tr_sp.at[idxr.at[p]], bufr[p], semr[p])
          for p in range(2)] + [None]
    gc = [pltpu.async_copy(tc_sp.at[idxc.at[p]], bufc[p], semc[p])
          for p in range(2)] + [None]
    oc = [None, None, None]
    for j in range(NCHUNK):
        p = j % 3
        gr[p].wait()
        gc[p].wait()

        def _add_row(r, _, p=p):
            for c in range(D_OUT // 16):
                sl = pl.ds(c * 16, 16)
                bufr[p][r, sl] = bufr[p][r, sl] + bufc[p][r, sl]
            return 0

        lax.fori_loop(0, CHUNK, _add_row, 0)
        oc[p] = pltpu.async_copy(
            bufr[p], out_hbm.at[pl.ds(base + j * CHUNK, CHUNK)], semo[p])
        if j + 2 < NCHUNK:
            q = (j + 2) % 3
            if oc[q] is not None:
                oc[q].wait()
            gr[q] = pltpu.async_copy(tr_sp.at[idxr.at[j + 2]], bufr[q], semr[q])
            gc[q] = pltpu.async_copy(tc_sp.at[idxc.at[j + 2]], bufc[q], semc[q])
    for p in range(3):
        if oc[p] is not None:
            oc[p].wait()


def kernel(coords, row_embed, col_embed, W, b):
    batch, n_spots, _ = coords.shape
    tr, tc = _make_tables(row_embed, col_embed, W, b)
    cc = jnp.clip(coords.astype(jnp.int32), 0, GRID - 1)
    rows = cc[..., 0].reshape(NW, NCHUNK, CHUNK)
    cols = cc[..., 1].reshape(NW, NCHUNK, CHUNK)
    out = _sc_lookup(tr, tc, rows, cols)
    return out.reshape(batch, n_spots, D_OUT)


# speedup vs baseline: 2.8217x; 1.0511x over previous
import functools

import jax
import jax.numpy as jnp
from jax import lax
from jax.experimental import pallas as pl
from jax.experimental.pallas import tpu as pltpu
from jax.experimental.pallas import tpu_sc as plsc

D_OUT = 128
HALF = 64
GRID = 256
NC, NS = 2, 16
NW = NC * NS
SPOTS = 16 * 4096
PER_W = SPOTS // NW
CHUNK = 128
NCHUNK = PER_W // CHUNK


def _tables_body(row_ref, col_ref, w_ref, b_ref, tr_ref, tc_ref):
    w = w_ref[...]
    tr = lax.dot_general(row_ref[...], w[:, :HALF],
                         (((1,), (1,)), ((), ())),
                         preferred_element_type=jnp.float32)
    tr_ref[...] = tr + b_ref[...]
    tc_ref[...] = lax.dot_general(col_ref[...], w[:, HALF:],
                                  (((1,), (1,)), ((), ())),
                                  preferred_element_type=jnp.float32)


def _make_tables(row_embed, col_embed, w, b):
    return pl.pallas_call(
        _tables_body,
        out_shape=(
            jax.ShapeDtypeStruct((GRID, D_OUT), jnp.float32),
            jax.ShapeDtypeStruct((GRID, D_OUT), jnp.float32),
        ),
    )(row_embed, col_embed, w, b.reshape(1, D_OUT))


_sc_mesh = plsc.VectorSubcoreMesh(core_axis_name="c", subcore_axis_name="s")


@functools.partial(
    pl.kernel,
    out_type=jax.ShapeDtypeStruct((SPOTS, D_OUT), jnp.float32),
    mesh=_sc_mesh,
    scratch_types=[
        pltpu.VMEM((NCHUNK, CHUNK), jnp.int32),
        pltpu.VMEM((NCHUNK, CHUNK), jnp.int32),
        pltpu.VMEM((CHUNK, D_OUT), jnp.float32),
        pltpu.VMEM((CHUNK, D_OUT), jnp.float32),
        pltpu.VMEM((CHUNK, D_OUT), jnp.float32),
        pltpu.VMEM((CHUNK, D_OUT), jnp.float32),
        pltpu.VMEM_SHARED((GRID, D_OUT), jnp.float32),
        pltpu.VMEM_SHARED((GRID, D_OUT), jnp.float32),
        pltpu.SemaphoreType.DMA,
        pltpu.SemaphoreType.DMA,
        pltpu.SemaphoreType.DMA,
        pltpu.SemaphoreType.DMA,
        pltpu.SemaphoreType.DMA,
        pltpu.SemaphoreType.DMA,
        pltpu.SemaphoreType.DMA,
        pltpu.SemaphoreType.DMA,
        pltpu.SemaphoreType.DMA,
        pltpu.SemaphoreType.DMA,
        pltpu.SemaphoreType.DMA,
        pltpu.SemaphoreType.DMA,
    ],
)
def _sc_lookup(tr_hbm, tc_hbm, rows_hbm, cols_hbm, out_hbm,
               idxr, idxc, buf0, buf1, buf2, buf3,
               tr_sp, tc_sp, semr0, semr1, semr2, semr3,
               sema0, sema1, sema2, sema3, semo0, semo1, semo2, semo3):
    wid = lax.axis_index("s") * NC + lax.axis_index("c")
    base = wid * PER_W
    buf = (buf0, buf1, buf2, buf3)
    semr = (semr0, semr1, semr2, semr3)
    sema = (sema0, sema1, sema2, sema3)
    semo = (semo0, semo1, semo2, semo3)
    @pl.when(lax.axis_index("s") == 0)
    def _stage():
        pltpu.sync_copy(tr_hbm, tr_sp)
        pltpu.sync_copy(tc_hbm, tc_sp)

    pltpu.sync_copy(rows_hbm.at[wid], idxr)
    pltpu.sync_copy(cols_hbm.at[wid], idxc)
    plsc.subcore_barrier()
    gr = [pltpu.async_copy(tr_sp.at[idxr.at[x]], buf[x], semr[x])
          for x in range(3)] + [None]
    gr[0].wait()
    ga = [pltpu.async_copy(tc_sp.at[idxc.at[0]], buf[0], sema[0], add=True),
          None, None, None]
    oc = [None, None, None, None]
    for j in range(NCHUNK):
        p = j % 4
        ga[p].wait()
        oc[p] = pltpu.async_copy(
            buf[p], out_hbm.at[pl.ds(base + j * CHUNK, CHUNK)], semo[p])
        if j + 1 < NCHUNK:
            pn = (j + 1) % 4
            gr[pn].wait()
            ga[pn] = pltpu.async_copy(tc_sp.at[idxc.at[j + 1]], buf[pn],
                                      sema[pn], add=True)
        if j + 3 < NCHUNK:
            q = (j + 3) % 4
            if oc[q] is not None:
                oc[q].wait()
            gr[q] = pltpu.async_copy(tr_sp.at[idxr.at[j + 3]], buf[q], semr[q])
    for p in range(4):
        if oc[p] is not None:
            oc[p].wait()


def kernel(coords, row_embed, col_embed, W, b):
    batch, n_spots, _ = coords.shape
    tr, tc = _make_tables(row_embed, col_embed, W, b)
    cc = jnp.clip(coords.astype(jnp.int32), 0, GRID - 1)
    rows = cc[..., 0].reshape(NW, NCHUNK, CHUNK)
    cols = cc[..., 1].reshape(NW, NCHUNK, CHUNK)
    out = _sc_lookup(tr, tc, rows, cols)
    return out.reshape(batch, n_spots, D_OUT)


# speedup vs baseline: 2.8339x; 1.0043x over previous
import functools

import jax
import jax.numpy as jnp
from jax import lax
from jax.experimental import pallas as pl
from jax.experimental.pallas import tpu as pltpu
from jax.experimental.pallas import tpu_sc as plsc

D_OUT = 128
HALF = 64
GRID = 256
NC, NS = 2, 16
NW = NC * NS
SPOTS = 16 * 4096
PER_W = SPOTS // NW
CHUNK = 128
NCHUNK = PER_W // CHUNK


def _tables_body(row_ref, col_ref, w_ref, b_ref, tr_ref, tc_ref):
    w = w_ref[...]
    tr = lax.dot_general(row_ref[...], w[:, :HALF],
                         (((1,), (1,)), ((), ())),
                         preferred_element_type=jnp.float32)
    tr_ref[...] = tr + b_ref[...]
    tc_ref[...] = lax.dot_general(col_ref[...], w[:, HALF:],
                                  (((1,), (1,)), ((), ())),
                                  preferred_element_type=jnp.float32)


def _make_tables(row_embed, col_embed, w, b):
    return pl.pallas_call(
        _tables_body,
        out_shape=(
            jax.ShapeDtypeStruct((GRID, D_OUT), jnp.float32),
            jax.ShapeDtypeStruct((GRID, D_OUT), jnp.float32),
        ),
    )(row_embed, col_embed, w, b.reshape(1, D_OUT))


_sc_mesh = plsc.VectorSubcoreMesh(core_axis_name="c", subcore_axis_name="s")


@functools.partial(
    pl.kernel,
    out_type=jax.ShapeDtypeStruct((SPOTS, D_OUT), jnp.float32),
    mesh=_sc_mesh,
    scratch_types=[
        pltpu.VMEM((NCHUNK, CHUNK), jnp.int32),
        pltpu.VMEM((NCHUNK, CHUNK), jnp.int32),
        pltpu.VMEM((CHUNK, D_OUT), jnp.float32),
        pltpu.VMEM((CHUNK, D_OUT), jnp.float32),
        pltpu.VMEM((CHUNK, D_OUT), jnp.float32),
        pltpu.VMEM((CHUNK, D_OUT), jnp.float32),
        pltpu.VMEM_SHARED((GRID, D_OUT), jnp.float32),
        pltpu.VMEM_SHARED((GRID, D_OUT), jnp.float32),
        pltpu.SemaphoreType.DMA,
        pltpu.SemaphoreType.DMA,
        pltpu.SemaphoreType.DMA,
        pltpu.SemaphoreType.DMA,
        pltpu.SemaphoreType.DMA,
        pltpu.SemaphoreType.DMA,
        pltpu.SemaphoreType.DMA,
        pltpu.SemaphoreType.DMA,
        pltpu.SemaphoreType.DMA,
        pltpu.SemaphoreType.DMA,
        pltpu.SemaphoreType.DMA,
        pltpu.SemaphoreType.DMA,
    ],
)
def _sc_lookup(tr_hbm, tc_hbm, rows_hbm, cols_hbm, out_hbm,
               idxr, idxc, buf0, buf1, buf2, buf3,
               tr_sp, tc_sp, semr0, semr1, semr2, semr3,
               sema0, sema1, sema2, sema3, semo0, semo1, semo2, semo3):
    wid = lax.axis_index("s") * NC + lax.axis_index("c")
    base = wid * PER_W
    buf = (buf0, buf1, buf2, buf3)
    semr = (semr0, semr1, semr2, semr3)
    sema = (sema0, sema1, sema2, sema3)
    semo = (semo0, semo1, semo2, semo3)
    @pl.when(lax.axis_index("s") == 0)
    def _stage():
        pltpu.sync_copy(tr_hbm, tr_sp)
        pltpu.sync_copy(tc_hbm, tc_sp)

    pltpu.sync_copy(rows_hbm.at[wid], idxr)
    pltpu.sync_copy(cols_hbm.at[wid], idxc)
    plsc.subcore_barrier()
    gr = [pltpu.async_copy(tr_sp.at[idxr.at[x]], buf[x], semr[x])
          for x in range(3)] + [None]
    ga = [None, None, None, None]
    for x in range(2):
        gr[x].wait()
        ga[x] = pltpu.async_copy(tc_sp.at[idxc.at[x]], buf[x], sema[x],
                                 add=True)
    oc = [None, None, None, None]
    for j in range(NCHUNK):
        p = j % 4
        ga[p].wait()
        oc[p] = pltpu.async_copy(
            buf[p], out_hbm.at[pl.ds(base + j * CHUNK, CHUNK)], semo[p])
        if j + 2 < NCHUNK:
            pn = (j + 2) % 4
            gr[pn].wait()
            ga[pn] = pltpu.async_copy(tc_sp.at[idxc.at[j + 2]], buf[pn],
                                      sema[pn], add=True)
        if j + 3 < NCHUNK:
            q = (j + 3) % 4
            if oc[q] is not None:
                oc[q].wait()
            gr[q] = pltpu.async_copy(tr_sp.at[idxr.at[j + 3]], buf[q], semr[q])
    for p in range(4):
        if oc[p] is not None:
            oc[p].wait()


def kernel(coords, row_embed, col_embed, W, b):
    batch, n_spots, _ = coords.shape
    tr, tc = _make_tables(row_embed, col_embed, W, b)
    cc = jnp.clip(coords.astype(jnp.int32), 0, GRID - 1)
    rows = cc[..., 0].reshape(NW, NCHUNK, CHUNK)
    cols = cc[..., 1].reshape(NW, NCHUNK, CHUNK)
    out = _sc_lookup(tr, tc, rows, cols)
    return out.reshape(batch, n_spots, D_OUT)
